# Initial kernel scaffold; baseline (speedup 1.0000x reference)
import jax
import jax.numpy as jnp
from jax.experimental import pallas as pl


def kernel(x, span_indices, W_width):
    raise NotImplementedError("write your pallas kernel here")


# speedup vs baseline: 8.3563x; 8.3563x over previous
import jax
import jax.numpy as jnp
from jax import lax
from jax.experimental import pallas as pl
from jax.experimental.pallas import tpu as pltpu
from jax.experimental.pallas import tpu_sc as plsc

_NC = 2
_NS = 16
_NW = _NC * _NS

_CHUNK = 32
_LANES = 16


def _make_gather(n_rows, seq_len, dim, rows_per_batch):
    rows_per_w = n_rows // _NW
    n_chunks = rows_per_w // _CHUNK
    workers_per_batch = rows_per_batch // rows_per_w
    mesh = plsc.VectorSubcoreMesh(core_axis_name="c", subcore_axis_name="s")

    def body(x_hbm, idx_hbm, out_hbm, idx_v, buf, sem):
        wid = lax.axis_index("s") * _NC + lax.axis_index("c")
        base = wid * rows_per_w
        offset = (wid // workers_per_batch) * seq_len

        pltpu.sync_copy(idx_hbm.at[wid], idx_v)
        for r in range(n_chunks):
            for j in range(_CHUNK // _LANES):
                sl = pl.ds(j * _LANES, _LANES)
                idx_v[r, sl] = idx_v[r, sl] + offset

        for c in range(n_chunks):
            pltpu.async_copy(x_hbm.at[idx_v.at[c]], buf, sem).wait()
            pltpu.sync_copy(buf, out_hbm.at[pl.ds(base + c * _CHUNK, _CHUNK)])

    return pl.kernel(
        body,
        out_type=jax.ShapeDtypeStruct((n_rows, dim), jnp.float32),
        mesh=mesh,
        scratch_types=[
            pltpu.VMEM((n_chunks, _CHUNK), jnp.int32),
            pltpu.VMEM((_CHUNK, dim), jnp.float32),
            pltpu.SemaphoreType.DMA,
        ],
    )


def kernel(x, span_indices, W_width):
    bs, seq_len, dim = x.shape
    n_spans = span_indices.shape[1]
    n_rows = bs * n_spans * 2

    x_flat = x.reshape(bs * seq_len, dim)
    idx_flat = span_indices.astype(jnp.int32).reshape(
        _NW, (n_rows // _NW) // _CHUNK, _CHUNK)

    out = _make_gather(n_rows, seq_len, dim, n_spans * 2)(x_flat, idx_flat)
    return out.reshape(bs, n_spans, 2 * dim)


# speedup vs baseline: 8.9688x; 1.0733x over previous
import jax
import jax.numpy as jnp
from jax import lax
from jax.experimental import pallas as pl
from jax.experimental.pallas import tpu as pltpu
from jax.experimental.pallas import tpu_sc as plsc

_NC = 2
_NS = 16
_NW = _NC * _NS

_CHUNK = 32
_LANES = 16


def _make_gather(n_rows, seq_len, dim, rows_per_batch):
    rows_per_w = n_rows // _NW
    n_chunks = rows_per_w // _CHUNK
    workers_per_batch = rows_per_batch // rows_per_w
    mesh = plsc.VectorSubcoreMesh(core_axis_name="c", subcore_axis_name="s")

    nbuf = 2

    def body(x_hbm, idx_hbm, out_hbm, idx_v, bufs, gsems, ssems):
        wid = lax.axis_index("s") * _NC + lax.axis_index("c")
        base = wid * rows_per_w
        offset = (wid // workers_per_batch) * seq_len

        pltpu.sync_copy(idx_hbm.at[wid], idx_v)
        for r in range(n_chunks):
            for j in range(_CHUNK // _LANES):
                sl = pl.ds(j * _LANES, _LANES)
                idx_v[r, sl] = idx_v[r, sl] + offset

        def gather(c):
            return pltpu.async_copy(
                x_hbm.at[idx_v.at[c]], bufs.at[c % nbuf], gsems.at[c % nbuf])

        def scatter(c):
            return pltpu.async_copy(
                bufs.at[c % nbuf],
                out_hbm.at[pl.ds(base + c * _CHUNK, _CHUNK)],
                ssems.at[c % nbuf])

        g = gather(0)
        scat = [None] * n_chunks
        for c in range(n_chunks):
            g_next = None
            if c + 1 < n_chunks:
                if c + 1 >= nbuf:
                    scat[c + 1 - nbuf].wait()
                g_next = gather(c + 1)
            g.wait()
            scat[c] = scatter(c)
            g = g_next
        for c in range(max(0, n_chunks - nbuf), n_chunks):
            scat[c].wait()

    return pl.kernel(
        body,
        out_type=jax.ShapeDtypeStruct((n_rows, dim), jnp.float32),
        mesh=mesh,
        scratch_types=[
            pltpu.VMEM((n_chunks, _CHUNK), jnp.int32),
            pltpu.VMEM((nbuf, _CHUNK, dim), jnp.float32),
            pltpu.SemaphoreType.DMA((nbuf,)),
            pltpu.SemaphoreType.DMA((nbuf,)),
        ],
    )


def kernel(x, span_indices, W_width):
    bs, seq_len, dim = x.shape
    n_spans = span_indices.shape[1]
    n_rows = bs * n_spans * 2

    x_flat = x.reshape(bs * seq_len, dim)
    idx_flat = span_indices.astype(jnp.int32).reshape(
        _NW, (n_rows // _NW) // _CHUNK, _CHUNK)

    out = _make_gather(n_rows, seq_len, dim, n_spans * 2)(x_flat, idx_flat)
    return out.reshape(bs, n_spans, 2 * dim)


# speedup vs baseline: 18.6313x; 2.0773x over previous
import jax
import jax.numpy as jnp
from jax import lax
from jax.experimental import pallas as pl
from jax.experimental.pallas import tpu as pltpu
from jax.experimental.pallas import tpu_sc as plsc

_NC = 2
_NS = 16
_NW = _NC * _NS

_CHUNK = 16
_LANES = 16


def _make_gather(bs, n_spans, seq_len, dim):
    spans_per_w = (bs * n_spans) // _NW
    n_chunks = spans_per_w // _CHUNK
    workers_per_batch = _NW // bs
    nbuf = 2
    mesh = plsc.VectorSubcoreMesh(core_axis_name="c", subcore_axis_name="s")

    def body(x_hbm, idx_hbm, out_hbm, idx_v, bufs, gsems, ssems):
        wid = lax.axis_index("s") * _NC + lax.axis_index("c")
        b = wid // workers_per_batch
        span0 = (wid % workers_per_batch) * spans_per_w
        offset = b * seq_len

        pltpu.sync_copy(idx_hbm.at[wid], idx_v)
        for r in range(n_chunks):
            for k in range(2):
                idx_v[r, k, :] = idx_v[r, k, :] + offset

        def gather(c):
            buf = bufs.at[c % nbuf]
            sem = gsems.at[c % nbuf]
            gb = pltpu.async_copy(
                x_hbm.at[idx_v.at[c, 0]], buf.at[:, pl.ds(0, dim)], sem)
            ge = pltpu.async_copy(
                x_hbm.at[idx_v.at[c, 1]], buf.at[:, pl.ds(dim, dim)], sem)
            return gb, ge

        def scatter(c):
            return pltpu.async_copy(
                bufs.at[c % nbuf],
                out_hbm.at[b, pl.ds(span0 + c * _CHUNK, _CHUNK)],
                ssems.at[c % nbuf])

        g = gather(0)
        scat = [None] * n_chunks
        for c in range(n_chunks):
            g_next = None
            if c + 1 < n_chunks:
                if c + 1 >= nbuf:
                    scat[c + 1 - nbuf].wait()
                g_next = gather(c + 1)
            g[0].wait()
            g[1].wait()
            scat[c] = scatter(c)
            g = g_next
        for c in range(max(0, n_chunks - nbuf), n_chunks):
            scat[c].wait()

    return pl.kernel(
        body,
        out_type=jax.ShapeDtypeStruct((bs, n_spans, 2 * dim), jnp.float32),
        mesh=mesh,
        scratch_types=[
            pltpu.VMEM((n_chunks, 2, _CHUNK), jnp.int32),
            pltpu.VMEM((nbuf, _CHUNK, 2 * dim), jnp.float32),
            pltpu.SemaphoreType.DMA((nbuf,)),
            pltpu.SemaphoreType.DMA((nbuf,)),
        ],
    )


def kernel(x, span_indices, W_width):
    bs, seq_len, dim = x.shape
    n_spans = span_indices.shape[1]
    spans_per_w = (bs * n_spans) // _NW
    n_chunks = spans_per_w // _CHUNK

    x_flat = x.reshape(bs * seq_len, dim)
    idx = span_indices.astype(jnp.int32).reshape(
        _NW, n_chunks, _CHUNK, 2).transpose(0, 1, 3, 2)

    out = _make_gather(bs, n_spans, seq_len, dim)(x_flat, idx)
    return out
